# Initial kernel scaffold; baseline (speedup 1.0000x reference)
import jax
import jax.numpy as jnp
from jax.experimental import pallas as pl


def kernel(x, edge_index, W1, b1, W2, b2, fcW1, fcb1, fcW2, fcb2):
    raise NotImplementedError("write your pallas kernel here")


# speedup vs baseline: 17.0420x; 17.0420x over previous
import functools

import jax
import jax.numpy as jnp
from jax import lax
from jax.experimental import pallas as pl
from jax.experimental.pallas import tpu as pltpu
from jax.experimental.pallas import tpu_sc as plsc

N = 50000
E = 800000
IN_DIM = 60
HID = 64
HHALF = HID // 2

NPAD = 50176
RT = NPAD // 16
ET = E // 16
EW = E // 32
KC = 400
NCH = ET // KC
ZR = 392

RB = 512
NB = NPAD // RB

_mesh = plsc.VectorSubcoreMesh(core_axis_name="c", subcore_axis_name="s")


def _deg_body(dst_hbm, ones_hbm, out_hbm, idx_v, ones_v, zbuf_v, acc_sh):
    c = lax.axis_index("c")
    s = lax.axis_index("s")
    w = c * 16 + s

    def _zf(i, _):
        zbuf_v[pl.ds(i * 16, 16)] = jnp.zeros((16,), jnp.float32)
        return 0

    lax.fori_loop(0, RT // 16, _zf, 0)
    pltpu.sync_copy(zbuf_v, acc_sh.at[pl.ds(s * RT, RT)])
    plsc.subcore_barrier()

    pltpu.sync_copy(ones_hbm, ones_v)
    pltpu.sync_copy(dst_hbm.at[pl.ds(w * EW, EW)], idx_v)
    pltpu.sync_copy(ones_v, acc_sh.at[idx_v], add=True)
    plsc.subcore_barrier()

    pltpu.sync_copy(acc_sh.at[pl.ds(s * RT, RT)], zbuf_v)
    pltpu.sync_copy(zbuf_v, out_hbm.at[pl.ds(c * NPAD + s * RT, RT)])


_deg = functools.partial(
    pl.kernel,
    out_type=jax.ShapeDtypeStruct((2 * NPAD,), jnp.float32),
    mesh=_mesh,
    scratch_types=[
        pltpu.VMEM((EW,), jnp.int32),
        pltpu.VMEM((EW,), jnp.float32),
        pltpu.VMEM((RT,), jnp.float32),
        pltpu.VMEM_SHARED((NPAD,), jnp.float32),
    ],
)(_deg_body)


def _agg_body(g_hbm, src2_hbm, dst_hbm, out_hbm, si_v, di_v, rows_v, sem, acc_sh):
    c = lax.axis_index("c")
    s = lax.axis_index("s")

    def _zf(i, _):
        rows_v[i, pl.ds(0, 16)] = jnp.zeros((16,), jnp.float32)
        rows_v[i, pl.ds(16, 16)] = jnp.zeros((16,), jnp.float32)
        return 0

    lax.fori_loop(0, KC, _zf, 0)

    def _zc(j, _):
        pltpu.sync_copy(rows_v.at[pl.ds(0, ZR)],
                        acc_sh.at[pl.ds(s * RT + j * ZR, ZR)])
        return 0

    lax.fori_loop(0, RT // ZR, _zc, 0)
    plsc.subcore_barrier()

    def _chunk(j, _):
        e0 = s * ET + j * KC
        pltpu.sync_copy(src2_hbm.at[pl.ds(c * E + e0, KC)], si_v)
        pltpu.sync_copy(dst_hbm.at[pl.ds(e0, KC)], di_v)
        pltpu.async_copy(g_hbm.at[si_v], rows_v, sem).wait()
        pltpu.sync_copy(rows_v, acc_sh.at[di_v], add=True)
        return 0

    lax.fori_loop(0, NCH, _chunk, 0)
    plsc.subcore_barrier()

    def _dc(j, _):
        pltpu.sync_copy(acc_sh.at[pl.ds(s * RT + j * ZR, ZR)],
                        rows_v.at[pl.ds(0, ZR)])
        pltpu.sync_copy(rows_v.at[pl.ds(0, ZR)],
                        out_hbm.at[pl.ds(c * NPAD + s * RT + j * ZR, ZR)])
        return 0

    lax.fori_loop(0, RT // ZR, _dc, 0)


_agg = functools.partial(
    pl.kernel,
    out_type=jax.ShapeDtypeStruct((2 * NPAD, HHALF), jnp.float32),
    mesh=_mesh,
    compiler_params=pltpu.CompilerParams(use_tc_tiling_on_sc=False),
    scratch_types=[
        pltpu.VMEM((KC,), jnp.int32),
        pltpu.VMEM((KC,), jnp.int32),
        pltpu.VMEM((KC, HHALF), jnp.float32),
        pltpu.SemaphoreType.DMA,
        pltpu.VMEM_SHARED((NPAD, HHALF), jnp.float32),
    ],
)(_agg_body)


def _mm1_body(x_ref, w_ref, d0_ref, d1_ref, g_ref, dinv_ref):
    deg = d0_ref[...] + d1_ref[...] + 1.0
    dinv = lax.rsqrt(deg)
    dinv_ref[...] = dinv
    h = jnp.dot(x_ref[...], w_ref[0], preferred_element_type=jnp.float32)
    g_ref[...] = h * dinv


def _mm1(x_pad, W1st, degp):
    return pl.pallas_call(
        _mm1_body,
        grid=(NB, 2),
        in_specs=[
            pl.BlockSpec((RB, IN_DIM), lambda i, c: (i, 0)),
            pl.BlockSpec((1, IN_DIM, HHALF), lambda i, c: (c, 0, 0)),
            pl.BlockSpec((RB, 1), lambda i, c: (i, 0)),
            pl.BlockSpec((RB, 1), lambda i, c: (NB + i, 0)),
        ],
        out_specs=[
            pl.BlockSpec((RB, HHALF), lambda i, c: (c * NB + i, 0)),
            pl.BlockSpec((RB, 1), lambda i, c: (i, 0)),
        ],
        out_shape=[
            jax.ShapeDtypeStruct((2 * NPAD, HHALF), jnp.float32),
            jax.ShapeDtypeStruct((NPAD, 1), jnp.float32),
        ],
    )(x_pad, W1st, degp, degp)


def _mid_body(sl_ref, sh_ref, gl_ref, gh_ref, dinv_ref, b1_ref, w2_ref, g2_ref):
    a = jnp.concatenate(
        [sl_ref[...] + gl_ref[...], sh_ref[...] + gh_ref[...]], axis=1)
    h = jnp.maximum(a * dinv_ref[...] + b1_ref[...], 0.0)
    g2_ref[...] = jnp.dot(
        h, w2_ref[0], preferred_element_type=jnp.float32) * dinv_ref[...]


def _mid(S1, g1, dinv, b1r, W2st):
    return pl.pallas_call(
        _mid_body,
        grid=(NB, 2),
        in_specs=[
            pl.BlockSpec((RB, HHALF), lambda i, c: (i, 0)),
            pl.BlockSpec((RB, HHALF), lambda i, c: (NB + i, 0)),
            pl.BlockSpec((RB, HHALF), lambda i, c: (i, 0)),
            pl.BlockSpec((RB, HHALF), lambda i, c: (NB + i, 0)),
            pl.BlockSpec((RB, 1), lambda i, c: (i, 0)),
            pl.BlockSpec((1, HID), lambda i, c: (0, 0)),
            pl.BlockSpec((1, HID, HHALF), lambda i, c: (c, 0, 0)),
        ],
        out_specs=pl.BlockSpec((RB, HHALF), lambda i, c: (c * NB + i, 0)),
        out_shape=jax.ShapeDtypeStruct((2 * NPAD, HHALF), jnp.float32),
    )(S1, S1, g1, g1, dinv, b1r, W2st)


def _out_body(sl_ref, sh_ref, gl_ref, gh_ref, dinv_ref, b2_ref, fw1_ref,
              fb1_ref, fw2_ref, fb2_ref, out_ref, h_ref):
    a = jnp.concatenate(
        [sl_ref[...] + gl_ref[...], sh_ref[...] + gh_ref[...]], axis=1)
    h = jnp.maximum(a * dinv_ref[...] + b2_ref[...], 0.0)
    h_ref[...] = h
    t = jnp.dot(h, fw1_ref[...], preferred_element_type=jnp.float32)
    t = jnp.maximum(t + fb1_ref[...], 0.0)
    out_ref[...] = jnp.dot(
        t, fw2_ref[...], preferred_element_type=jnp.float32) + fb2_ref[...]


def _out(S2, g2, dinv, b2r, fcW1, fcb1r, fcW2, fcb2r):
    return pl.pallas_call(
        _out_body,
        grid=(NB,),
        in_specs=[
            pl.BlockSpec((RB, HHALF), lambda i: (i, 0)),
            pl.BlockSpec((RB, HHALF), lambda i: (NB + i, 0)),
            pl.BlockSpec((RB, HHALF), lambda i: (i, 0)),
            pl.BlockSpec((RB, HHALF), lambda i: (NB + i, 0)),
            pl.BlockSpec((RB, 1), lambda i: (i, 0)),
            pl.BlockSpec((1, HID), lambda i: (0, 0)),
            pl.BlockSpec((HID, HHALF), lambda i: (0, 0)),
            pl.BlockSpec((1, HHALF), lambda i: (0, 0)),
            pl.BlockSpec((HHALF, 1), lambda i: (0, 0)),
            pl.BlockSpec((1, 1), lambda i: (0, 0)),
        ],
        out_specs=[
            pl.BlockSpec((RB, 1), lambda i: (i, 0)),
            pl.BlockSpec((RB, HID), lambda i: (i, 0)),
        ],
        out_shape=[
            jax.ShapeDtypeStruct((NPAD, 1), jnp.float32),
            jax.ShapeDtypeStruct((NPAD, HID), jnp.float32),
        ],
    )(S2, S2, g2, g2, dinv, b2r, fcW1, fcb1r, fcW2, fcb2r)


def kernel(x, edge_index, W1, b1, W2, b2, fcW1, fcb1, fcW2, fcb2):
    src = edge_index[0].astype(jnp.int32)
    dst = edge_index[1].astype(jnp.int32)
    src2 = jnp.concatenate([src, src + NPAD])
    ones = jnp.ones((EW,), jnp.float32)

    x_pad = jnp.pad(x, ((0, NPAD - N), (0, 0)))
    W1st = jnp.stack([W1[:, :HHALF], W1[:, HHALF:]])
    W2st = jnp.stack([W2[:, :HHALF], W2[:, HHALF:]])
    degp = _deg(dst, ones).reshape(2 * NPAD, 1)

    g1, dinv = _mm1(x_pad, W1st, degp)
    S1 = _agg(g1, src2, dst)
    g2 = _mid(S1, g1, dinv, b1.reshape(1, HID), W2st)
    S2 = _agg(g2, src2, dst)
    out_p, h_p = _out(S2, g2, dinv, b2.reshape(1, HID), fcW1,
                      fcb1.reshape(1, HHALF), fcW2, fcb2.reshape(1, 1))
    return out_p[:N], h_p[:N]
